# baseline (speedup 1.0000x reference)
import jax
import jax.numpy as jnp
from jax import lax
from jax.experimental import pallas as pl
from jax.experimental.pallas import tpu as pltpu
from jax.experimental.pallas import tpu_sc as plsc

N = 10000
E = 320000
D = 128

NC, NS = 2, 16
NW = NC * NS
CHUNK = 64
NCHUNK = 160
EP = NW * NCHUNK * CHUNK
TRASH = N
R_DEG = 10240
RPT_DEG = R_DEG // NS
R_ACC = 10112
RPT_ACC = R_ACC // NS
BR = 1024
NBUF = 2
NROUND = NCHUNK // NBUF

_mesh = plsc.VectorSubcoreMesh(core_axis_name="c", subcore_axis_name="s")


def _deg_body(dst3, degp, idx_v, ones_v, zrow_v, deg_acc):
  c = lax.axis_index("c")
  s = lax.axis_index("s")
  zeros16 = jnp.zeros((16,), jnp.float32)
  ones16 = jnp.ones((16,), jnp.float32)

  def fill_z(i, _):
    zrow_v[pl.ds(i * 16, 16)] = zeros16
    return 0
  lax.fori_loop(0, RPT_DEG // 16, fill_z, 0)
  for i in range(CHUNK // 16):
    ones_v[pl.ds(i * 16, 16)] = ones16

  pltpu.sync_copy(zrow_v, deg_acc.at[pl.ds(s * RPT_DEG, RPT_DEG)])
  plsc.subcore_barrier()

  w = s * NC + c
  pltpu.sync_copy(dst3.at[w], idx_v)

  def scat(j, _):
    pltpu.sync_copy(ones_v, deg_acc.at[idx_v.at[j]], add=True)
    return 0
  lax.fori_loop(0, NCHUNK, scat, 0)

  plsc.subcore_barrier()
  pltpu.sync_copy(deg_acc.at[pl.ds(s * RPT_DEG, RPT_DEG)],
                  degp.at[c, pl.ds(s * RPT_DEG, RPT_DEG)])


_sc_deg = pl.kernel(
    _deg_body,
    out_type=jax.ShapeDtypeStruct((NC, R_DEG), jnp.float32),
    mesh=_mesh,
    scratch_types=[
        pltpu.VMEM((NCHUNK, CHUNK), jnp.int32),
        pltpu.VMEM((CHUNK,), jnp.float32),
        pltpu.VMEM((RPT_DEG,), jnp.float32),
        pltpu.VMEM_SHARED((R_DEG,), jnp.float32),
    ],
)


def _scat_body(g, src3, dst3, parts, src_v, dst_v, buf0, buf1,
               sem0, sem1, acc):
  bufs = (buf0, buf1)
  sems = (sem0, sem1)
  c = lax.axis_index("c")
  s = lax.axis_index("s")
  zeros16 = jnp.zeros((16,), jnp.float32)

  def zrow(i, _):
    for k in range(D // 16):
      bufs[0][i, pl.ds(k * 16, 16)] = zeros16
    return 0
  lax.fori_loop(0, CHUNK, zrow, 0)
  nz = RPT_ACC // CHUNK
  rz = RPT_ACC - nz * CHUNK
  for j in range(nz):
    pltpu.sync_copy(bufs[0], acc.at[pl.ds(s * RPT_ACC + j * CHUNK, CHUNK)])
  pltpu.sync_copy(bufs[0].at[pl.ds(0, rz)],
                  acc.at[pl.ds(s * RPT_ACC + nz * CHUNK, rz)])

  w = s * NC + c
  pltpu.sync_copy(src3.at[w], src_v)
  pltpu.sync_copy(dst3.at[w], dst_v)
  plsc.subcore_barrier()

  def round_(iv, _):
    descs = [pltpu.async_copy(g.at[src_v.at[iv * NBUF + b]], bufs[b],
                              sems[b]) for b in range(NBUF)]
    for b in range(NBUF):
      descs[b].wait()
      pltpu.sync_copy(bufs[b], acc.at[dst_v.at[iv * NBUF + b]], add=True)
    return 0
  lax.fori_loop(0, NROUND, round_, 0)

  plsc.subcore_barrier()
  pltpu.sync_copy(acc.at[pl.ds(s * RPT_ACC, RPT_ACC)],
                  parts.at[c, pl.ds(s * RPT_ACC, RPT_ACC)])


_sc_scatter = pl.kernel(
    _scat_body,
    out_type=jax.ShapeDtypeStruct((NC, R_ACC, D), jnp.float32),
    mesh=_mesh,
    scratch_types=[
        pltpu.VMEM((NCHUNK, CHUNK), jnp.int32),
        pltpu.VMEM((NCHUNK, CHUNK), jnp.int32),
        pltpu.VMEM((CHUNK, D), jnp.float32),
        pltpu.VMEM((CHUNK, D), jnp.float32),
        pltpu.SemaphoreType.DMA,
        pltpu.SemaphoreType.DMA,
        pltpu.VMEM_SHARED((R_ACC, D), jnp.float32),
    ],
    compiler_params=pltpu.CompilerParams(use_tc_tiling_on_sc=False),
)


def _mid_body(x_ref, w_ref, b_ref, degp_ref, g_ref, selfb_ref):
  h = jnp.dot(x_ref[...], w_ref[...], preferred_element_type=jnp.float32)
  deg = degp_ref[0, :] + degp_ref[1, :] + 1.0
  dis = lax.rsqrt(deg)
  g_ref[...] = h * dis[:, None]
  selfb_ref[...] = h * (1.0 / deg)[:, None] + b_ref[...]


def _tc_mid(x, W, b2, degp):
  return pl.pallas_call(
      _mid_body,
      grid=((N + BR - 1) // BR,),
      in_specs=[
          pl.BlockSpec((BR, D), lambda i: (i, 0)),
          pl.BlockSpec((D, D), lambda i: (0, 0)),
          pl.BlockSpec((1, D), lambda i: (0, 0)),
          pl.BlockSpec((NC, BR), lambda i: (0, i)),
      ],
      out_specs=[
          pl.BlockSpec((BR, D), lambda i: (i, 0)),
          pl.BlockSpec((BR, D), lambda i: (i, 0)),
      ],
      out_shape=[
          jax.ShapeDtypeStruct((N, D), jnp.float32),
          jax.ShapeDtypeStruct((N, D), jnp.float32),
      ],
  )(x, W, b2, degp)


def _final_body(parts_ref, degp_ref, selfb_ref, out_ref):
  deg = degp_ref[0, :] + degp_ref[1, :] + 1.0
  dis = lax.rsqrt(deg)
  psum = parts_ref[0] + parts_ref[1]
  out_ref[...] = selfb_ref[...] + psum * dis[:, None]


def _tc_final(parts, degp, selfb):
  return pl.pallas_call(
      _final_body,
      grid=((N + BR - 1) // BR,),
      in_specs=[
          pl.BlockSpec((NC, BR, D), lambda i: (0, i, 0)),
          pl.BlockSpec((NC, BR), lambda i: (0, i)),
          pl.BlockSpec((BR, D), lambda i: (i, 0)),
      ],
      out_specs=pl.BlockSpec((BR, D), lambda i: (i, 0)),
      out_shape=jax.ShapeDtypeStruct((N, D), jnp.float32),
  )(parts, degp, selfb)


@jax.jit
def kernel(x, edge_index, W, b):
  src = edge_index[0].astype(jnp.int32)
  dst = edge_index[1].astype(jnp.int32)
  pad = EP - E
  src3 = jnp.concatenate([src, jnp.zeros((pad,), jnp.int32)]).reshape(
      NW, NCHUNK, CHUNK)
  dst3 = jnp.concatenate([dst, jnp.full((pad,), TRASH, jnp.int32)]).reshape(
      NW, NCHUNK, CHUNK)
  degp = _sc_deg(dst3)
  g, selfb = _tc_mid(x, W, b.reshape(1, D), degp)
  parts = _sc_scatter(g, src3, dst3)
  return _tc_final(parts, degp, selfb)


# speedup vs baseline: 1.0379x; 1.0379x over previous
import jax
import jax.numpy as jnp
from jax import lax
from jax.experimental import pallas as pl
from jax.experimental.pallas import tpu as pltpu
from jax.experimental.pallas import tpu_sc as plsc

N = 10000
E = 320000
D = 128

NC, NS = 2, 16
NW = NC * NS
CHUNK = 80
NCHUNK = 128
EP = NW * NCHUNK * CHUNK
TRASH = N
R_DEG = 10240
RPT_DEG = R_DEG // NS
R_ACC = 10112
RPT_ACC = R_ACC // NS
BR = 1024
NBUF = 2
NROUND = NCHUNK // NBUF

SRC_BITS = 14
SRC_MASK = (1 << SRC_BITS) - 1

_mesh = plsc.VectorSubcoreMesh(core_axis_name="c", subcore_axis_name="s")


def _deg_body(dst3, degp, idx_v, ones_v, zrow_v, deg_acc):
  c = lax.axis_index("c")
  s = lax.axis_index("s")
  zeros16 = jnp.zeros((16,), jnp.float32)
  ones16 = jnp.ones((16,), jnp.float32)

  def fill_z(i, _):
    zrow_v[pl.ds(i * 16, 16)] = zeros16
    return 0
  lax.fori_loop(0, RPT_DEG // 16, fill_z, 0)
  for i in range(CHUNK // 16):
    ones_v[pl.ds(i * 16, 16)] = ones16

  pltpu.sync_copy(zrow_v, deg_acc.at[pl.ds(s * RPT_DEG, RPT_DEG)])
  plsc.subcore_barrier()

  w = s * NC + c
  pltpu.sync_copy(dst3.at[w], idx_v)

  def scat(j, _):
    pltpu.sync_copy(ones_v, deg_acc.at[idx_v.at[j]], add=True)
    return 0
  lax.fori_loop(0, NCHUNK, scat, 0)

  plsc.subcore_barrier()
  pltpu.sync_copy(deg_acc.at[pl.ds(s * RPT_DEG, RPT_DEG)],
                  degp.at[c, pl.ds(s * RPT_DEG, RPT_DEG)])


_sc_deg = pl.kernel(
    _deg_body,
    out_type=jax.ShapeDtypeStruct((NC, R_DEG), jnp.float32),
    mesh=_mesh,
    scratch_types=[
        pltpu.VMEM((NCHUNK, CHUNK), jnp.int32),
        pltpu.VMEM((CHUNK,), jnp.float32),
        pltpu.VMEM((RPT_DEG,), jnp.float32),
        pltpu.VMEM_SHARED((R_DEG,), jnp.float32),
    ],
)


def _scat_body(g, src3, dst3, parts, src_v, dst_v, buf0, buf1,
               sem0, sem1, acc):
  bufs = (buf0, buf1)
  sems = (sem0, sem1)
  c = lax.axis_index("c")
  s = lax.axis_index("s")
  zeros16 = jnp.zeros((16,), jnp.float32)

  def zrow(i, _):
    for k in range(D // 16):
      bufs[0][i, pl.ds(k * 16, 16)] = zeros16
    return 0
  lax.fori_loop(0, CHUNK, zrow, 0)
  nz = RPT_ACC // CHUNK
  rz = RPT_ACC - nz * CHUNK
  for j in range(nz):
    pltpu.sync_copy(bufs[0], acc.at[pl.ds(s * RPT_ACC + j * CHUNK, CHUNK)])
  pltpu.sync_copy(bufs[0].at[pl.ds(0, rz)],
                  acc.at[pl.ds(s * RPT_ACC + nz * CHUNK, rz)])

  w = s * NC + c
  pltpu.sync_copy(src3.at[w], src_v)
  pltpu.sync_copy(dst3.at[w], dst_v)
  plsc.subcore_barrier()

  def round_(iv, _):
    descs = [pltpu.async_copy(g.at[src_v.at[iv * NBUF + b]], bufs[b],
                              sems[b]) for b in range(NBUF)]
    for b in range(NBUF):
      descs[b].wait()
      pltpu.sync_copy(bufs[b], acc.at[dst_v.at[iv * NBUF + b]], add=True)
    return 0
  lax.fori_loop(0, NROUND, round_, 0)

  plsc.subcore_barrier()
  pltpu.sync_copy(acc.at[pl.ds(s * RPT_ACC, RPT_ACC)],
                  parts.at[c, pl.ds(s * RPT_ACC, RPT_ACC)])


_sc_scatter = pl.kernel(
    _scat_body,
    out_type=jax.ShapeDtypeStruct((NC, R_ACC, D), jnp.float32),
    mesh=_mesh,
    scratch_types=[
        pltpu.VMEM((NCHUNK, CHUNK), jnp.int32),
        pltpu.VMEM((NCHUNK, CHUNK), jnp.int32),
        pltpu.VMEM((CHUNK, D), jnp.float32),
        pltpu.VMEM((CHUNK, D), jnp.float32),
        pltpu.SemaphoreType.DMA,
        pltpu.SemaphoreType.DMA,
        pltpu.VMEM_SHARED((R_ACC, D), jnp.float32),
    ],
    compiler_params=pltpu.CompilerParams(use_tc_tiling_on_sc=False),
)


def _mid_body(x_ref, w_ref, b_ref, degp_ref, g_ref, selfb_ref):
  h = jnp.dot(x_ref[...], w_ref[...], preferred_element_type=jnp.float32)
  deg = degp_ref[0, :] + degp_ref[1, :] + 1.0
  dis = lax.rsqrt(deg)
  g_ref[...] = h * dis[:, None]
  selfb_ref[...] = h * (1.0 / deg)[:, None] + b_ref[...]


def _tc_mid(x, W, b2, degp):
  return pl.pallas_call(
      _mid_body,
      grid=((N + BR - 1) // BR,),
      in_specs=[
          pl.BlockSpec((BR, D), lambda i: (i, 0)),
          pl.BlockSpec((D, D), lambda i: (0, 0)),
          pl.BlockSpec((1, D), lambda i: (0, 0)),
          pl.BlockSpec((NC, BR), lambda i: (0, i)),
      ],
      out_specs=[
          pl.BlockSpec((BR, D), lambda i: (i, 0)),
          pl.BlockSpec((BR, D), lambda i: (i, 0)),
      ],
      out_shape=[
          jax.ShapeDtypeStruct((N, D), jnp.float32),
          jax.ShapeDtypeStruct((N, D), jnp.float32),
      ],
  )(x, W, b2, degp)


def _final_body(parts_ref, degp_ref, selfb_ref, out_ref):
  deg = degp_ref[0, :] + degp_ref[1, :] + 1.0
  dis = lax.rsqrt(deg)
  psum = parts_ref[0] + parts_ref[1]
  out_ref[...] = selfb_ref[...] + psum * dis[:, None]


def _tc_final(parts, degp, selfb):
  return pl.pallas_call(
      _final_body,
      grid=((N + BR - 1) // BR,),
      in_specs=[
          pl.BlockSpec((NC, BR, D), lambda i: (0, i, 0)),
          pl.BlockSpec((NC, BR), lambda i: (0, i)),
          pl.BlockSpec((BR, D), lambda i: (i, 0)),
      ],
      out_specs=pl.BlockSpec((BR, D), lambda i: (i, 0)),
      out_shape=jax.ShapeDtypeStruct((N, D), jnp.float32),
  )(parts, degp, selfb)


@jax.jit
def kernel(x, edge_index, W, b):
  src = edge_index[0].astype(jnp.int32)
  dst = edge_index[1].astype(jnp.int32)
  pad = EP - E
  src3 = jnp.concatenate([src, jnp.zeros((pad,), jnp.int32)]).reshape(
      NW, NCHUNK, CHUNK)
  dst3 = jnp.concatenate([dst, jnp.full((pad,), TRASH, jnp.int32)]).reshape(
      NW, NCHUNK, CHUNK)
  degp = _sc_deg(dst3)
  g, selfb = _tc_mid(x, W, b.reshape(1, D), degp)
  parts = _sc_scatter(g, src3, dst3)
  return _tc_final(parts, degp, selfb)


# speedup vs baseline: 1.2048x; 1.1608x over previous
import jax
import jax.numpy as jnp
from jax import lax
from jax.experimental import pallas as pl
from jax.experimental.pallas import tpu as pltpu
from jax.experimental.pallas import tpu_sc as plsc

N = 10000
E = 320000
D = 128

NC, NS = 2, 16
NW = NC * NS
CHUNK = 80
NCHUNK = 128
NCHT = NC * NS * NCHUNK
CF = 0
NF = 192
NSC = 64
NT_ROWS = NF * NS + NSC * NS
EP = NT_ROWS * CHUNK
TRASH = N
R_DEG = 10240
RPT_DEG = R_DEG // NS
R_ACC = 10112
RPT_ACC = R_ACC // NS
BR = 1024
NBUF = 2
NROUND = NCHUNK // NBUF

SRC_BITS = 14
SRC_MASK = (1 << SRC_BITS) - 1

_mesh = plsc.VectorSubcoreMesh(core_axis_name="c", subcore_axis_name="s")


def _deg_body(dst3, degp, idx_v, ones_v, zrow_v, deg_acc):
  c = lax.axis_index("c")
  s = lax.axis_index("s")
  zeros16 = jnp.zeros((16,), jnp.float32)
  ones16 = jnp.ones((16,), jnp.float32)

  def fill_z(i, _):
    zrow_v[pl.ds(i * 16, 16)] = zeros16
    return 0
  lax.fori_loop(0, RPT_DEG // 16, fill_z, 0)
  for i in range(CHUNK // 16):
    ones_v[pl.ds(i * 16, 16)] = ones16

  pltpu.sync_copy(zrow_v, deg_acc.at[pl.ds(s * RPT_DEG, RPT_DEG)])
  plsc.subcore_barrier()

  w = s * NC + c
  pltpu.sync_copy(dst3.at[w], idx_v)

  def scat(j, _):
    pltpu.sync_copy(ones_v, deg_acc.at[idx_v.at[j]], add=True)
    return 0
  lax.fori_loop(0, NCHUNK, scat, 0)

  plsc.subcore_barrier()
  pltpu.sync_copy(deg_acc.at[pl.ds(s * RPT_DEG, RPT_DEG)],
                  degp.at[c, pl.ds(s * RPT_DEG, RPT_DEG)])


_sc_deg = pl.kernel(
    _deg_body,
    out_type=jax.ShapeDtypeStruct((NC, R_DEG), jnp.float32),
    mesh=_mesh,
    scratch_types=[
        pltpu.VMEM((NCHUNK, CHUNK), jnp.int32),
        pltpu.VMEM((CHUNK,), jnp.float32),
        pltpu.VMEM((RPT_DEG,), jnp.float32),
        pltpu.VMEM_SHARED((R_DEG,), jnp.float32),
    ],
)


def _scat_body(g, src2, dst2, parts, src_v, dst_v, buf0, buf1,
               sem0, sem1, acc):
  bufs = (buf0, buf1)
  sems = (sem0, sem1)
  c = lax.axis_index("c")
  s = lax.axis_index("s")
  zeros16 = jnp.zeros((16,), jnp.float32)

  def zrow(i, _):
    for k in range(D // 16):
      bufs[0][i, pl.ds(k * 16, 16)] = zeros16
    return 0
  lax.fori_loop(0, CHUNK, zrow, 0)
  nz = RPT_ACC // CHUNK
  rz = RPT_ACC - nz * CHUNK
  for j in range(nz):
    pltpu.sync_copy(bufs[0], acc.at[pl.ds(s * RPT_ACC + j * CHUNK, CHUNK)])
  pltpu.sync_copy(bufs[0].at[pl.ds(0, rz)],
                  acc.at[pl.ds(s * RPT_ACC + nz * CHUNK, rz)])

  is_fast = (c == CF)
  plsc.subcore_barrier()

  def round_(iv, _):
    descs = [pltpu.async_copy(g.at[src_v.at[iv * NBUF + b]], bufs[b],
                              sems[b]) for b in range(NBUF)]
    for b in range(NBUF):
      descs[b].wait()
      pltpu.sync_copy(bufs[b], acc.at[dst_v.at[iv * NBUF + b]], add=True)
    return 0

  st0 = jnp.where(is_fast, s * NF, NS * NF + s * NSC)
  @pl.when(is_fast)
  def _():
    pltpu.sync_copy(src2.at[pl.ds(st0, 128)], src_v)
    pltpu.sync_copy(dst2.at[pl.ds(st0, 128)], dst_v)
  @pl.when(jnp.logical_not(is_fast))
  def _():
    pltpu.sync_copy(src2.at[pl.ds(st0, NSC)], src_v.at[pl.ds(0, NSC)])
    pltpu.sync_copy(dst2.at[pl.ds(st0, NSC)], dst_v.at[pl.ds(0, NSC)])
  n0 = jnp.where(is_fast, 128, NSC)
  lax.fori_loop(0, n0 // NBUF, round_, 0)

  @pl.when(is_fast)
  def _():
    st1 = s * NF + 128
    pltpu.sync_copy(src2.at[pl.ds(st1, NF - 128)], src_v.at[pl.ds(0, NF - 128)])
    pltpu.sync_copy(dst2.at[pl.ds(st1, NF - 128)], dst_v.at[pl.ds(0, NF - 128)])
  n1 = jnp.where(is_fast, NF - 128, 0)
  lax.fori_loop(0, n1 // NBUF, round_, 0)

  plsc.subcore_barrier()
  pltpu.sync_copy(acc.at[pl.ds(s * RPT_ACC, RPT_ACC)],
                  parts.at[c, pl.ds(s * RPT_ACC, RPT_ACC)])


_sc_scatter = pl.kernel(
    _scat_body,
    out_type=jax.ShapeDtypeStruct((NC, R_ACC, D), jnp.float32),
    mesh=_mesh,
    scratch_types=[
        pltpu.VMEM((128, CHUNK), jnp.int32),
        pltpu.VMEM((128, CHUNK), jnp.int32),
        pltpu.VMEM((CHUNK, D), jnp.float32),
        pltpu.VMEM((CHUNK, D), jnp.float32),
        pltpu.SemaphoreType.DMA,
        pltpu.SemaphoreType.DMA,
        pltpu.VMEM_SHARED((R_ACC, D), jnp.float32),
    ],
    compiler_params=pltpu.CompilerParams(use_tc_tiling_on_sc=False),
)


def _mid_body(x_ref, w_ref, b_ref, degp_ref, g_ref, selfb_ref):
  h = jnp.dot(x_ref[...], w_ref[...], preferred_element_type=jnp.float32)
  deg = degp_ref[0, :] + degp_ref[1, :] + 1.0
  dis = lax.rsqrt(deg)
  g_ref[...] = h * dis[:, None]
  selfb_ref[...] = h * (1.0 / deg)[:, None] + b_ref[...]


def _tc_mid(x, W, b2, degp):
  return pl.pallas_call(
      _mid_body,
      grid=((N + BR - 1) // BR,),
      in_specs=[
          pl.BlockSpec((BR, D), lambda i: (i, 0)),
          pl.BlockSpec((D, D), lambda i: (0, 0)),
          pl.BlockSpec((1, D), lambda i: (0, 0)),
          pl.BlockSpec((NC, BR), lambda i: (0, i)),
      ],
      out_specs=[
          pl.BlockSpec((BR, D), lambda i: (i, 0)),
          pl.BlockSpec((BR, D), lambda i: (i, 0)),
      ],
      out_shape=[
          jax.ShapeDtypeStruct((N, D), jnp.float32),
          jax.ShapeDtypeStruct((N, D), jnp.float32),
      ],
  )(x, W, b2, degp)


def _final_body(parts_ref, degp_ref, selfb_ref, out_ref):
  deg = degp_ref[0, :] + degp_ref[1, :] + 1.0
  dis = lax.rsqrt(deg)
  psum = parts_ref[0] + parts_ref[1]
  out_ref[...] = selfb_ref[...] + psum * dis[:, None]


def _tc_final(parts, degp, selfb):
  return pl.pallas_call(
      _final_body,
      grid=((N + BR - 1) // BR,),
      in_specs=[
          pl.BlockSpec((NC, BR, D), lambda i: (0, i, 0)),
          pl.BlockSpec((NC, BR), lambda i: (0, i)),
          pl.BlockSpec((BR, D), lambda i: (i, 0)),
      ],
      out_specs=pl.BlockSpec((BR, D), lambda i: (i, 0)),
      out_shape=jax.ShapeDtypeStruct((N, D), jnp.float32),
  )(parts, degp, selfb)


@jax.jit
def kernel(x, edge_index, W, b):
  src = edge_index[0].astype(jnp.int32)
  dst = edge_index[1].astype(jnp.int32)
  pad = EP - E
  src2 = jnp.concatenate([src, jnp.zeros((pad,), jnp.int32)]).reshape(
      NT_ROWS, CHUNK)
  dst2 = jnp.concatenate([dst, jnp.full((pad,), TRASH, jnp.int32)]).reshape(
      NT_ROWS, CHUNK)
  degp = _sc_deg(dst2.reshape(NT_ROWS // NCHUNK, NCHUNK, CHUNK))
  g, selfb = _tc_mid(x, W, b.reshape(1, D), degp)
  parts = _sc_scatter(g, src2, dst2)
  return _tc_final(parts, degp, selfb)


# speedup vs baseline: 1.2477x; 1.0356x over previous
import jax
import jax.numpy as jnp
from jax import lax
from jax.experimental import pallas as pl
from jax.experimental.pallas import tpu as pltpu
from jax.experimental.pallas import tpu_sc as plsc

N = 10000
E = 320000
D = 128

NC, NS = 2, 16
NW = NC * NS
CHUNK = 80
NCHUNK = 128
NCHT = NC * NS * NCHUNK
CF = 1
NF = 192
NSC = 64
NT_ROWS = NF * NS + NSC * NS
EP = NT_ROWS * CHUNK
TRASH = N
R_DEG = 10240
RPT_DEG = R_DEG // NS
R_ACC = 10112
RPT_ACC = R_ACC // NS
BR = 1024
NBUF = 2
NROUND = NCHUNK // NBUF

SRC_BITS = 14
SRC_MASK = (1 << SRC_BITS) - 1

_mesh = plsc.VectorSubcoreMesh(core_axis_name="c", subcore_axis_name="s")


def _deg_body(dst3, degp, idx_v, ones_v, zrow_v, deg_acc):
  c = lax.axis_index("c")
  s = lax.axis_index("s")
  zeros16 = jnp.zeros((16,), jnp.float32)
  ones16 = jnp.ones((16,), jnp.float32)

  def fill_z(i, _):
    zrow_v[pl.ds(i * 16, 16)] = zeros16
    return 0
  lax.fori_loop(0, RPT_DEG // 16, fill_z, 0)
  for i in range(CHUNK // 16):
    ones_v[pl.ds(i * 16, 16)] = ones16

  pltpu.sync_copy(zrow_v, deg_acc.at[pl.ds(s * RPT_DEG, RPT_DEG)])
  plsc.subcore_barrier()

  w = s * NC + c
  pltpu.sync_copy(dst3.at[w], idx_v)

  def scat(j, _):
    pltpu.sync_copy(ones_v, deg_acc.at[idx_v.at[j]], add=True)
    return 0
  lax.fori_loop(0, NCHUNK, scat, 0)

  plsc.subcore_barrier()
  pltpu.sync_copy(deg_acc.at[pl.ds(s * RPT_DEG, RPT_DEG)],
                  degp.at[c, pl.ds(s * RPT_DEG, RPT_DEG)])


_sc_deg = pl.kernel(
    _deg_body,
    out_type=jax.ShapeDtypeStruct((NC, R_DEG), jnp.float32),
    mesh=_mesh,
    scratch_types=[
        pltpu.VMEM((NCHUNK, CHUNK), jnp.int32),
        pltpu.VMEM((CHUNK,), jnp.float32),
        pltpu.VMEM((RPT_DEG,), jnp.float32),
        pltpu.VMEM_SHARED((R_DEG,), jnp.float32),
    ],
)


def _scat_body(g, src2, dst2, parts, src_v, dst_v, buf0, buf1,
               sem0, sem1, acc):
  bufs = (buf0, buf1)
  sems = (sem0, sem1)
  c = lax.axis_index("c")
  s = lax.axis_index("s")
  zeros16 = jnp.zeros((16,), jnp.float32)

  def zrow(i, _):
    for k in range(D // 16):
      bufs[0][i, pl.ds(k * 16, 16)] = zeros16
    return 0
  lax.fori_loop(0, CHUNK, zrow, 0)
  nz = RPT_ACC // CHUNK
  rz = RPT_ACC - nz * CHUNK
  for j in range(nz):
    pltpu.sync_copy(bufs[0], acc.at[pl.ds(s * RPT_ACC + j * CHUNK, CHUNK)])
  pltpu.sync_copy(bufs[0].at[pl.ds(0, rz)],
                  acc.at[pl.ds(s * RPT_ACC + nz * CHUNK, rz)])

  is_fast = (c == CF)
  plsc.subcore_barrier()

  def round_(iv, _):
    descs = [pltpu.async_copy(g.at[src_v.at[iv * NBUF + b]], bufs[b],
                              sems[b]) for b in range(NBUF)]
    for b in range(NBUF):
      descs[b].wait()
      pltpu.sync_copy(bufs[b], acc.at[dst_v.at[iv * NBUF + b]], add=True)
    return 0

  st0 = jnp.where(is_fast, s * NF, NS * NF + s * NSC)
  @pl.when(is_fast)
  def _():
    pltpu.sync_copy(src2.at[pl.ds(st0, 128)], src_v)
    pltpu.sync_copy(dst2.at[pl.ds(st0, 128)], dst_v)
  @pl.when(jnp.logical_not(is_fast))
  def _():
    pltpu.sync_copy(src2.at[pl.ds(st0, NSC)], src_v.at[pl.ds(0, NSC)])
    pltpu.sync_copy(dst2.at[pl.ds(st0, NSC)], dst_v.at[pl.ds(0, NSC)])
  n0 = jnp.where(is_fast, 128, NSC)
  lax.fori_loop(0, n0 // NBUF, round_, 0)

  @pl.when(is_fast)
  def _():
    st1 = s * NF + 128
    pltpu.sync_copy(src2.at[pl.ds(st1, NF - 128)], src_v.at[pl.ds(0, NF - 128)])
    pltpu.sync_copy(dst2.at[pl.ds(st1, NF - 128)], dst_v.at[pl.ds(0, NF - 128)])
  n1 = jnp.where(is_fast, NF - 128, 0)
  lax.fori_loop(0, n1 // NBUF, round_, 0)

  plsc.subcore_barrier()
  pltpu.sync_copy(acc.at[pl.ds(s * RPT_ACC, RPT_ACC)],
                  parts.at[c, pl.ds(s * RPT_ACC, RPT_ACC)])


_sc_scatter = pl.kernel(
    _scat_body,
    out_type=jax.ShapeDtypeStruct((NC, R_ACC, D), jnp.float32),
    mesh=_mesh,
    scratch_types=[
        pltpu.VMEM((128, CHUNK), jnp.int32),
        pltpu.VMEM((128, CHUNK), jnp.int32),
        pltpu.VMEM((CHUNK, D), jnp.float32),
        pltpu.VMEM((CHUNK, D), jnp.float32),
        pltpu.SemaphoreType.DMA,
        pltpu.SemaphoreType.DMA,
        pltpu.VMEM_SHARED((R_ACC, D), jnp.float32),
    ],
    compiler_params=pltpu.CompilerParams(use_tc_tiling_on_sc=False),
)


def _mid_body(x_ref, w_ref, b_ref, degp_ref, g_ref, selfb_ref):
  h = jnp.dot(x_ref[...], w_ref[...], preferred_element_type=jnp.float32)
  deg = degp_ref[0, :] + degp_ref[1, :] + 1.0
  dis = lax.rsqrt(deg)
  g_ref[...] = h * dis[:, None]
  selfb_ref[...] = h * (1.0 / deg)[:, None] + b_ref[...]


def _tc_mid(x, W, b2, degp):
  return pl.pallas_call(
      _mid_body,
      grid=((N + BR - 1) // BR,),
      in_specs=[
          pl.BlockSpec((BR, D), lambda i: (i, 0)),
          pl.BlockSpec((D, D), lambda i: (0, 0)),
          pl.BlockSpec((1, D), lambda i: (0, 0)),
          pl.BlockSpec((NC, BR), lambda i: (0, i)),
      ],
      out_specs=[
          pl.BlockSpec((BR, D), lambda i: (i, 0)),
          pl.BlockSpec((BR, D), lambda i: (i, 0)),
      ],
      out_shape=[
          jax.ShapeDtypeStruct((N, D), jnp.float32),
          jax.ShapeDtypeStruct((N, D), jnp.float32),
      ],
  )(x, W, b2, degp)


def _final_body(parts_ref, degp_ref, selfb_ref, out_ref):
  deg = degp_ref[0, :] + degp_ref[1, :] + 1.0
  dis = lax.rsqrt(deg)
  psum = parts_ref[0] + parts_ref[1]
  out_ref[...] = selfb_ref[...] + psum * dis[:, None]


def _tc_final(parts, degp, selfb):
  return pl.pallas_call(
      _final_body,
      grid=((N + BR - 1) // BR,),
      in_specs=[
          pl.BlockSpec((NC, BR, D), lambda i: (0, i, 0)),
          pl.BlockSpec((NC, BR), lambda i: (0, i)),
          pl.BlockSpec((BR, D), lambda i: (i, 0)),
      ],
      out_specs=pl.BlockSpec((BR, D), lambda i: (i, 0)),
      out_shape=jax.ShapeDtypeStruct((N, D), jnp.float32),
  )(parts, degp, selfb)


@jax.jit
def kernel(x, edge_index, W, b):
  src = edge_index[0].astype(jnp.int32)
  dst = edge_index[1].astype(jnp.int32)
  pad = EP - E
  src2 = jnp.concatenate([src, jnp.zeros((pad,), jnp.int32)]).reshape(
      NT_ROWS, CHUNK)
  dst2 = jnp.concatenate([dst, jnp.full((pad,), TRASH, jnp.int32)]).reshape(
      NT_ROWS, CHUNK)
  degp = _sc_deg(dst2.reshape(NT_ROWS // NCHUNK, NCHUNK, CHUNK))
  g, selfb = _tc_mid(x, W, b.reshape(1, D), degp)
  parts = _sc_scatter(g, src2, dst2)
  return _tc_final(parts, degp, selfb)


# speedup vs baseline: 1.2978x; 1.0401x over previous
import jax
import jax.numpy as jnp
from jax import lax
from jax.experimental import pallas as pl
from jax.experimental.pallas import tpu as pltpu
from jax.experimental.pallas import tpu_sc as plsc

N = 10000
E = 320000
D = 128

NC, NS = 2, 16
NW = NC * NS
CHUNK = 80
NCHUNK = 128
NCHT = NC * NS * NCHUNK
CF = 1
NF = 208
NSC = 48
NT_ROWS = NF * NS + NSC * NS
EP = NT_ROWS * CHUNK
TRASH = N
R_DEG = 10240
RPT_DEG = R_DEG // NS
R_ACC = 10112
RPT_ACC = R_ACC // NS
BR = 1024
NBUF = 2
NROUND = NCHUNK // NBUF

SRC_BITS = 14
SRC_MASK = (1 << SRC_BITS) - 1

_mesh = plsc.VectorSubcoreMesh(core_axis_name="c", subcore_axis_name="s")


def _deg_body(dst3, degp, idx_v, ones_v, zrow_v, deg_acc):
  c = lax.axis_index("c")
  s = lax.axis_index("s")
  zeros16 = jnp.zeros((16,), jnp.float32)
  ones16 = jnp.ones((16,), jnp.float32)

  def fill_z(i, _):
    zrow_v[pl.ds(i * 16, 16)] = zeros16
    return 0
  lax.fori_loop(0, RPT_DEG // 16, fill_z, 0)
  for i in range(CHUNK // 16):
    ones_v[pl.ds(i * 16, 16)] = ones16

  pltpu.sync_copy(zrow_v, deg_acc.at[pl.ds(s * RPT_DEG, RPT_DEG)])
  plsc.subcore_barrier()

  w = s * NC + c
  pltpu.sync_copy(dst3.at[w], idx_v)

  def scat(j, _):
    pltpu.sync_copy(ones_v, deg_acc.at[idx_v.at[j]], add=True)
    return 0
  lax.fori_loop(0, NCHUNK, scat, 0)

  plsc.subcore_barrier()
  pltpu.sync_copy(deg_acc.at[pl.ds(s * RPT_DEG, RPT_DEG)],
                  degp.at[c, pl.ds(s * RPT_DEG, RPT_DEG)])


_sc_deg = pl.kernel(
    _deg_body,
    out_type=jax.ShapeDtypeStruct((NC, R_DEG), jnp.float32),
    mesh=_mesh,
    scratch_types=[
        pltpu.VMEM((NCHUNK, CHUNK), jnp.int32),
        pltpu.VMEM((CHUNK,), jnp.float32),
        pltpu.VMEM((RPT_DEG,), jnp.float32),
        pltpu.VMEM_SHARED((R_DEG,), jnp.float32),
    ],
)


def _scat_body(g, src2, dst2, parts, src_v, dst_v, buf0, buf1,
               sem0, sem1, acc):
  bufs = (buf0, buf1)
  sems = (sem0, sem1)
  c = lax.axis_index("c")
  s = lax.axis_index("s")
  zeros16 = jnp.zeros((16,), jnp.float32)

  def zrow(i, _):
    for k in range(D // 16):
      bufs[0][i, pl.ds(k * 16, 16)] = zeros16
    return 0
  lax.fori_loop(0, CHUNK, zrow, 0)
  nz = RPT_ACC // CHUNK
  rz = RPT_ACC - nz * CHUNK
  for j in range(nz):
    pltpu.sync_copy(bufs[0], acc.at[pl.ds(s * RPT_ACC + j * CHUNK, CHUNK)])
  pltpu.sync_copy(bufs[0].at[pl.ds(0, rz)],
                  acc.at[pl.ds(s * RPT_ACC + nz * CHUNK, rz)])

  is_fast = (c == CF)
  plsc.subcore_barrier()

  def round_(iv, _):
    descs = [pltpu.async_copy(g.at[src_v.at[iv * NBUF + b]], bufs[b],
                              sems[b]) for b in range(NBUF)]
    for b in range(NBUF):
      descs[b].wait()
      pltpu.sync_copy(bufs[b], acc.at[dst_v.at[iv * NBUF + b]], add=True)
    return 0

  st0 = jnp.where(is_fast, s * NF, NS * NF + s * NSC)
  @pl.when(is_fast)
  def _():
    pltpu.sync_copy(src2.at[pl.ds(st0, 128)], src_v)
    pltpu.sync_copy(dst2.at[pl.ds(st0, 128)], dst_v)
  @pl.when(jnp.logical_not(is_fast))
  def _():
    pltpu.sync_copy(src2.at[pl.ds(st0, NSC)], src_v.at[pl.ds(0, NSC)])
    pltpu.sync_copy(dst2.at[pl.ds(st0, NSC)], dst_v.at[pl.ds(0, NSC)])
  n0 = jnp.where(is_fast, 128, NSC)
  lax.fori_loop(0, n0 // NBUF, round_, 0)

  @pl.when(is_fast)
  def _():
    st1 = s * NF + 128
    pltpu.sync_copy(src2.at[pl.ds(st1, NF - 128)], src_v.at[pl.ds(0, NF - 128)])
    pltpu.sync_copy(dst2.at[pl.ds(st1, NF - 128)], dst_v.at[pl.ds(0, NF - 128)])
  n1 = jnp.where(is_fast, NF - 128, 0)
  lax.fori_loop(0, n1 // NBUF, round_, 0)

  plsc.subcore_barrier()
  pltpu.sync_copy(acc.at[pl.ds(s * RPT_ACC, RPT_ACC)],
                  parts.at[c, pl.ds(s * RPT_ACC, RPT_ACC)])


_sc_scatter = pl.kernel(
    _scat_body,
    out_type=jax.ShapeDtypeStruct((NC, R_ACC, D), jnp.float32),
    mesh=_mesh,
    scratch_types=[
        pltpu.VMEM((128, CHUNK), jnp.int32),
        pltpu.VMEM((128, CHUNK), jnp.int32),
        pltpu.VMEM((CHUNK, D), jnp.float32),
        pltpu.VMEM((CHUNK, D), jnp.float32),
        pltpu.SemaphoreType.DMA,
        pltpu.SemaphoreType.DMA,
        pltpu.VMEM_SHARED((R_ACC, D), jnp.float32),
    ],
    compiler_params=pltpu.CompilerParams(use_tc_tiling_on_sc=False),
)


def _mid_body(x_ref, w_ref, b_ref, degp_ref, g_ref, selfb_ref):
  h = jnp.dot(x_ref[...], w_ref[...], preferred_element_type=jnp.float32)
  deg = degp_ref[0, :] + degp_ref[1, :] + 1.0
  dis = lax.rsqrt(deg)
  g_ref[...] = h * dis[:, None]
  selfb_ref[...] = h * (1.0 / deg)[:, None] + b_ref[...]


def _tc_mid(x, W, b2, degp):
  return pl.pallas_call(
      _mid_body,
      grid=((N + BR - 1) // BR,),
      in_specs=[
          pl.BlockSpec((BR, D), lambda i: (i, 0)),
          pl.BlockSpec((D, D), lambda i: (0, 0)),
          pl.BlockSpec((1, D), lambda i: (0, 0)),
          pl.BlockSpec((NC, BR), lambda i: (0, i)),
      ],
      out_specs=[
          pl.BlockSpec((BR, D), lambda i: (i, 0)),
          pl.BlockSpec((BR, D), lambda i: (i, 0)),
      ],
      out_shape=[
          jax.ShapeDtypeStruct((N, D), jnp.float32),
          jax.ShapeDtypeStruct((N, D), jnp.float32),
      ],
  )(x, W, b2, degp)


def _final_body(parts_ref, degp_ref, selfb_ref, out_ref):
  deg = degp_ref[0, :] + degp_ref[1, :] + 1.0
  dis = lax.rsqrt(deg)
  psum = parts_ref[0] + parts_ref[1]
  out_ref[...] = selfb_ref[...] + psum * dis[:, None]


def _tc_final(parts, degp, selfb):
  return pl.pallas_call(
      _final_body,
      grid=((N + BR - 1) // BR,),
      in_specs=[
          pl.BlockSpec((NC, BR, D), lambda i: (0, i, 0)),
          pl.BlockSpec((NC, BR), lambda i: (0, i)),
          pl.BlockSpec((BR, D), lambda i: (i, 0)),
      ],
      out_specs=pl.BlockSpec((BR, D), lambda i: (i, 0)),
      out_shape=jax.ShapeDtypeStruct((N, D), jnp.float32),
  )(parts, degp, selfb)


@jax.jit
def kernel(x, edge_index, W, b):
  src = edge_index[0].astype(jnp.int32)
  dst = edge_index[1].astype(jnp.int32)
  pad = EP - E
  src2 = jnp.concatenate([src, jnp.zeros((pad,), jnp.int32)]).reshape(
      NT_ROWS, CHUNK)
  dst2 = jnp.concatenate([dst, jnp.full((pad,), TRASH, jnp.int32)]).reshape(
      NT_ROWS, CHUNK)
  degp = _sc_deg(dst2.reshape(NT_ROWS // NCHUNK, NCHUNK, CHUNK))
  g, selfb = _tc_mid(x, W, b.reshape(1, D), degp)
  parts = _sc_scatter(g, src2, dst2)
  return _tc_final(parts, degp, selfb)


# speedup vs baseline: 1.3571x; 1.0457x over previous
import jax
import jax.numpy as jnp
from jax import lax
from jax.experimental import pallas as pl
from jax.experimental.pallas import tpu as pltpu
from jax.experimental.pallas import tpu_sc as plsc

N = 10000
E = 320000
D = 128

NC, NS = 2, 16
NW = NC * NS
CHUNK = 80
NCHUNK = 128
NCHT = NC * NS * NCHUNK
CF = 1
NF = 224
NSC = 32
NT_ROWS = NF * NS + NSC * NS
EP = NT_ROWS * CHUNK
TRASH = N
R_DEG = 10240
RPT_DEG = R_DEG // NS
R_ACC = 10112
RPT_ACC = R_ACC // NS
BR = 1024
NBUF = 2
NROUND = NCHUNK // NBUF

SRC_BITS = 14
SRC_MASK = (1 << SRC_BITS) - 1

_mesh = plsc.VectorSubcoreMesh(core_axis_name="c", subcore_axis_name="s")


def _deg_body(dst3, degp, idx_v, ones_v, zrow_v, deg_acc):
  c = lax.axis_index("c")
  s = lax.axis_index("s")
  zeros16 = jnp.zeros((16,), jnp.float32)
  ones16 = jnp.ones((16,), jnp.float32)

  def fill_z(i, _):
    zrow_v[pl.ds(i * 16, 16)] = zeros16
    return 0
  lax.fori_loop(0, RPT_DEG // 16, fill_z, 0)
  for i in range(CHUNK // 16):
    ones_v[pl.ds(i * 16, 16)] = ones16

  pltpu.sync_copy(zrow_v, deg_acc.at[pl.ds(s * RPT_DEG, RPT_DEG)])
  plsc.subcore_barrier()

  w = s * NC + c
  pltpu.sync_copy(dst3.at[w], idx_v)

  def scat(j, _):
    pltpu.sync_copy(ones_v, deg_acc.at[idx_v.at[j]], add=True)
    return 0
  lax.fori_loop(0, NCHUNK, scat, 0)

  plsc.subcore_barrier()
  pltpu.sync_copy(deg_acc.at[pl.ds(s * RPT_DEG, RPT_DEG)],
                  degp.at[c, pl.ds(s * RPT_DEG, RPT_DEG)])


_sc_deg = pl.kernel(
    _deg_body,
    out_type=jax.ShapeDtypeStruct((NC, R_DEG), jnp.float32),
    mesh=_mesh,
    scratch_types=[
        pltpu.VMEM((NCHUNK, CHUNK), jnp.int32),
        pltpu.VMEM((CHUNK,), jnp.float32),
        pltpu.VMEM((RPT_DEG,), jnp.float32),
        pltpu.VMEM_SHARED((R_DEG,), jnp.float32),
    ],
)


def _scat_body(g, src2, dst2, parts, src_v, dst_v, buf0, buf1,
               sem0, sem1, acc):
  bufs = (buf0, buf1)
  sems = (sem0, sem1)
  c = lax.axis_index("c")
  s = lax.axis_index("s")
  zeros16 = jnp.zeros((16,), jnp.float32)

  def zrow(i, _):
    for k in range(D // 16):
      bufs[0][i, pl.ds(k * 16, 16)] = zeros16
    return 0
  lax.fori_loop(0, CHUNK, zrow, 0)
  nz = RPT_ACC // CHUNK
  rz = RPT_ACC - nz * CHUNK
  for j in range(nz):
    pltpu.sync_copy(bufs[0], acc.at[pl.ds(s * RPT_ACC + j * CHUNK, CHUNK)])
  pltpu.sync_copy(bufs[0].at[pl.ds(0, rz)],
                  acc.at[pl.ds(s * RPT_ACC + nz * CHUNK, rz)])

  is_fast = (c == CF)
  plsc.subcore_barrier()

  def round_(iv, _):
    descs = [pltpu.async_copy(g.at[src_v.at[iv * NBUF + b]], bufs[b],
                              sems[b]) for b in range(NBUF)]
    for b in range(NBUF):
      descs[b].wait()
      pltpu.sync_copy(bufs[b], acc.at[dst_v.at[iv * NBUF + b]], add=True)
    return 0

  st0 = jnp.where(is_fast, s * NF, NS * NF + s * NSC)
  @pl.when(is_fast)
  def _():
    pltpu.sync_copy(src2.at[pl.ds(st0, 128)], src_v)
    pltpu.sync_copy(dst2.at[pl.ds(st0, 128)], dst_v)
  @pl.when(jnp.logical_not(is_fast))
  def _():
    pltpu.sync_copy(src2.at[pl.ds(st0, NSC)], src_v.at[pl.ds(0, NSC)])
    pltpu.sync_copy(dst2.at[pl.ds(st0, NSC)], dst_v.at[pl.ds(0, NSC)])
  n0 = jnp.where(is_fast, 128, NSC)
  lax.fori_loop(0, n0 // NBUF, round_, 0)

  @pl.when(is_fast)
  def _():
    st1 = s * NF + 128
    pltpu.sync_copy(src2.at[pl.ds(st1, NF - 128)], src_v.at[pl.ds(0, NF - 128)])
    pltpu.sync_copy(dst2.at[pl.ds(st1, NF - 128)], dst_v.at[pl.ds(0, NF - 128)])
  n1 = jnp.where(is_fast, NF - 128, 0)
  lax.fori_loop(0, n1 // NBUF, round_, 0)

  plsc.subcore_barrier()
  pltpu.sync_copy(acc.at[pl.ds(s * RPT_ACC, RPT_ACC)],
                  parts.at[c, pl.ds(s * RPT_ACC, RPT_ACC)])


_sc_scatter = pl.kernel(
    _scat_body,
    out_type=jax.ShapeDtypeStruct((NC, R_ACC, D), jnp.float32),
    mesh=_mesh,
    scratch_types=[
        pltpu.VMEM((128, CHUNK), jnp.int32),
        pltpu.VMEM((128, CHUNK), jnp.int32),
        pltpu.VMEM((CHUNK, D), jnp.float32),
        pltpu.VMEM((CHUNK, D), jnp.float32),
        pltpu.SemaphoreType.DMA,
        pltpu.SemaphoreType.DMA,
        pltpu.VMEM_SHARED((R_ACC, D), jnp.float32),
    ],
    compiler_params=pltpu.CompilerParams(use_tc_tiling_on_sc=False),
)


def _mid_body(x_ref, w_ref, b_ref, degp_ref, g_ref, selfb_ref):
  h = jnp.dot(x_ref[...], w_ref[...], preferred_element_type=jnp.float32)
  deg = degp_ref[0, :] + degp_ref[1, :] + 1.0
  dis = lax.rsqrt(deg)
  g_ref[...] = h * dis[:, None]
  selfb_ref[...] = h * (1.0 / deg)[:, None] + b_ref[...]


def _tc_mid(x, W, b2, degp):
  return pl.pallas_call(
      _mid_body,
      grid=((N + BR - 1) // BR,),
      in_specs=[
          pl.BlockSpec((BR, D), lambda i: (i, 0)),
          pl.BlockSpec((D, D), lambda i: (0, 0)),
          pl.BlockSpec((1, D), lambda i: (0, 0)),
          pl.BlockSpec((NC, BR), lambda i: (0, i)),
      ],
      out_specs=[
          pl.BlockSpec((BR, D), lambda i: (i, 0)),
          pl.BlockSpec((BR, D), lambda i: (i, 0)),
      ],
      out_shape=[
          jax.ShapeDtypeStruct((N, D), jnp.float32),
          jax.ShapeDtypeStruct((N, D), jnp.float32),
      ],
  )(x, W, b2, degp)


def _final_body(parts_ref, degp_ref, selfb_ref, out_ref):
  deg = degp_ref[0, :] + degp_ref[1, :] + 1.0
  dis = lax.rsqrt(deg)
  psum = parts_ref[0] + parts_ref[1]
  out_ref[...] = selfb_ref[...] + psum * dis[:, None]


def _tc_final(parts, degp, selfb):
  return pl.pallas_call(
      _final_body,
      grid=((N + BR - 1) // BR,),
      in_specs=[
          pl.BlockSpec((NC, BR, D), lambda i: (0, i, 0)),
          pl.BlockSpec((NC, BR), lambda i: (0, i)),
          pl.BlockSpec((BR, D), lambda i: (i, 0)),
      ],
      out_specs=pl.BlockSpec((BR, D), lambda i: (i, 0)),
      out_shape=jax.ShapeDtypeStruct((N, D), jnp.float32),
  )(parts, degp, selfb)


@jax.jit
def kernel(x, edge_index, W, b):
  src = edge_index[0].astype(jnp.int32)
  dst = edge_index[1].astype(jnp.int32)
  pad = EP - E
  src2 = jnp.concatenate([src, jnp.zeros((pad,), jnp.int32)]).reshape(
      NT_ROWS, CHUNK)
  dst2 = jnp.concatenate([dst, jnp.full((pad,), TRASH, jnp.int32)]).reshape(
      NT_ROWS, CHUNK)
  degp = _sc_deg(dst2.reshape(NT_ROWS // NCHUNK, NCHUNK, CHUNK))
  g, selfb = _tc_mid(x, W, b.reshape(1, D), degp)
  parts = _sc_scatter(g, src2, dst2)
  return _tc_final(parts, degp, selfb)


# speedup vs baseline: 1.4023x; 1.0333x over previous
import jax
import jax.numpy as jnp
from jax import lax
from jax.experimental import pallas as pl
from jax.experimental.pallas import tpu as pltpu
from jax.experimental.pallas import tpu_sc as plsc

N = 10000
E = 320000
D = 128

NC, NS = 2, 16
NW = NC * NS
CHUNK = 80
NCHUNK = 128
NCHT = NC * NS * NCHUNK
CF = 1
NF = 240
NSC = 16
NT_ROWS = NF * NS + NSC * NS
EP = NT_ROWS * CHUNK
TRASH = N
R_DEG = 10240
RPT_DEG = R_DEG // NS
R_ACC = 10112
RPT_ACC = R_ACC // NS
BR = 1024
NBUF = 2
NROUND = NCHUNK // NBUF

SRC_BITS = 14
SRC_MASK = (1 << SRC_BITS) - 1

_mesh = plsc.VectorSubcoreMesh(core_axis_name="c", subcore_axis_name="s")


def _deg_body(dst3, degp, idx_v, ones_v, zrow_v, deg_acc):
  c = lax.axis_index("c")
  s = lax.axis_index("s")
  zeros16 = jnp.zeros((16,), jnp.float32)
  ones16 = jnp.ones((16,), jnp.float32)

  def fill_z(i, _):
    zrow_v[pl.ds(i * 16, 16)] = zeros16
    return 0
  lax.fori_loop(0, RPT_DEG // 16, fill_z, 0)
  for i in range(CHUNK // 16):
    ones_v[pl.ds(i * 16, 16)] = ones16

  pltpu.sync_copy(zrow_v, deg_acc.at[pl.ds(s * RPT_DEG, RPT_DEG)])
  plsc.subcore_barrier()

  w = s * NC + c
  pltpu.sync_copy(dst3.at[w], idx_v)

  def scat(j, _):
    pltpu.sync_copy(ones_v, deg_acc.at[idx_v.at[j]], add=True)
    return 0
  lax.fori_loop(0, NCHUNK, scat, 0)

  plsc.subcore_barrier()
  pltpu.sync_copy(deg_acc.at[pl.ds(s * RPT_DEG, RPT_DEG)],
                  degp.at[c, pl.ds(s * RPT_DEG, RPT_DEG)])


_sc_deg = pl.kernel(
    _deg_body,
    out_type=jax.ShapeDtypeStruct((NC, R_DEG), jnp.float32),
    mesh=_mesh,
    scratch_types=[
        pltpu.VMEM((NCHUNK, CHUNK), jnp.int32),
        pltpu.VMEM((CHUNK,), jnp.float32),
        pltpu.VMEM((RPT_DEG,), jnp.float32),
        pltpu.VMEM_SHARED((R_DEG,), jnp.float32),
    ],
)


def _scat_body(g, src2, dst2, parts, src_v, dst_v, buf0, buf1,
               sem0, sem1, acc):
  bufs = (buf0, buf1)
  sems = (sem0, sem1)
  c = lax.axis_index("c")
  s = lax.axis_index("s")
  zeros16 = jnp.zeros((16,), jnp.float32)

  def zrow(i, _):
    for k in range(D // 16):
      bufs[0][i, pl.ds(k * 16, 16)] = zeros16
    return 0
  lax.fori_loop(0, CHUNK, zrow, 0)
  nz = RPT_ACC // CHUNK
  rz = RPT_ACC - nz * CHUNK
  for j in range(nz):
    pltpu.sync_copy(bufs[0], acc.at[pl.ds(s * RPT_ACC + j * CHUNK, CHUNK)])
  pltpu.sync_copy(bufs[0].at[pl.ds(0, rz)],
                  acc.at[pl.ds(s * RPT_ACC + nz * CHUNK, rz)])

  is_fast = (c == CF)
  plsc.subcore_barrier()

  def round_(iv, _):
    descs = [pltpu.async_copy(g.at[src_v.at[iv * NBUF + b]], bufs[b],
                              sems[b]) for b in range(NBUF)]
    for b in range(NBUF):
      descs[b].wait()
      pltpu.sync_copy(bufs[b], acc.at[dst_v.at[iv * NBUF + b]], add=True)
    return 0

  st0 = jnp.where(is_fast, s * NF, NS * NF + s * NSC)
  @pl.when(is_fast)
  def _():
    pltpu.sync_copy(src2.at[pl.ds(st0, 128)], src_v)
    pltpu.sync_copy(dst2.at[pl.ds(st0, 128)], dst_v)
  @pl.when(jnp.logical_not(is_fast))
  def _():
    pltpu.sync_copy(src2.at[pl.ds(st0, NSC)], src_v.at[pl.ds(0, NSC)])
    pltpu.sync_copy(dst2.at[pl.ds(st0, NSC)], dst_v.at[pl.ds(0, NSC)])
  n0 = jnp.where(is_fast, 128, NSC)
  lax.fori_loop(0, n0 // NBUF, round_, 0)

  @pl.when(is_fast)
  def _():
    st1 = s * NF + 128
    pltpu.sync_copy(src2.at[pl.ds(st1, NF - 128)], src_v.at[pl.ds(0, NF - 128)])
    pltpu.sync_copy(dst2.at[pl.ds(st1, NF - 128)], dst_v.at[pl.ds(0, NF - 128)])
  n1 = jnp.where(is_fast, NF - 128, 0)
  lax.fori_loop(0, n1 // NBUF, round_, 0)

  plsc.subcore_barrier()
  pltpu.sync_copy(acc.at[pl.ds(s * RPT_ACC, RPT_ACC)],
                  parts.at[c, pl.ds(s * RPT_ACC, RPT_ACC)])


_sc_scatter = pl.kernel(
    _scat_body,
    out_type=jax.ShapeDtypeStruct((NC, R_ACC, D), jnp.float32),
    mesh=_mesh,
    scratch_types=[
        pltpu.VMEM((128, CHUNK), jnp.int32),
        pltpu.VMEM((128, CHUNK), jnp.int32),
        pltpu.VMEM((CHUNK, D), jnp.float32),
        pltpu.VMEM((CHUNK, D), jnp.float32),
        pltpu.SemaphoreType.DMA,
        pltpu.SemaphoreType.DMA,
        pltpu.VMEM_SHARED((R_ACC, D), jnp.float32),
    ],
    compiler_params=pltpu.CompilerParams(use_tc_tiling_on_sc=False),
)


def _mid_body(x_ref, w_ref, b_ref, degp_ref, g_ref, selfb_ref):
  h = jnp.dot(x_ref[...], w_ref[...], preferred_element_type=jnp.float32)
  deg = degp_ref[0, :] + degp_ref[1, :] + 1.0
  dis = lax.rsqrt(deg)
  g_ref[...] = h * dis[:, None]
  selfb_ref[...] = h * (1.0 / deg)[:, None] + b_ref[...]


def _tc_mid(x, W, b2, degp):
  return pl.pallas_call(
      _mid_body,
      grid=((N + BR - 1) // BR,),
      in_specs=[
          pl.BlockSpec((BR, D), lambda i: (i, 0)),
          pl.BlockSpec((D, D), lambda i: (0, 0)),
          pl.BlockSpec((1, D), lambda i: (0, 0)),
          pl.BlockSpec((NC, BR), lambda i: (0, i)),
      ],
      out_specs=[
          pl.BlockSpec((BR, D), lambda i: (i, 0)),
          pl.BlockSpec((BR, D), lambda i: (i, 0)),
      ],
      out_shape=[
          jax.ShapeDtypeStruct((N, D), jnp.float32),
          jax.ShapeDtypeStruct((N, D), jnp.float32),
      ],
  )(x, W, b2, degp)


def _final_body(parts_ref, degp_ref, selfb_ref, out_ref):
  deg = degp_ref[0, :] + degp_ref[1, :] + 1.0
  dis = lax.rsqrt(deg)
  psum = parts_ref[0] + parts_ref[1]
  out_ref[...] = selfb_ref[...] + psum * dis[:, None]


def _tc_final(parts, degp, selfb):
  return pl.pallas_call(
      _final_body,
      grid=((N + BR - 1) // BR,),
      in_specs=[
          pl.BlockSpec((NC, BR, D), lambda i: (0, i, 0)),
          pl.BlockSpec((NC, BR), lambda i: (0, i)),
          pl.BlockSpec((BR, D), lambda i: (i, 0)),
      ],
      out_specs=pl.BlockSpec((BR, D), lambda i: (i, 0)),
      out_shape=jax.ShapeDtypeStruct((N, D), jnp.float32),
  )(parts, degp, selfb)


@jax.jit
def kernel(x, edge_index, W, b):
  src = edge_index[0].astype(jnp.int32)
  dst = edge_index[1].astype(jnp.int32)
  pad = EP - E
  src2 = jnp.concatenate([src, jnp.zeros((pad,), jnp.int32)]).reshape(
      NT_ROWS, CHUNK)
  dst2 = jnp.concatenate([dst, jnp.full((pad,), TRASH, jnp.int32)]).reshape(
      NT_ROWS, CHUNK)
  degp = _sc_deg(dst2.reshape(NT_ROWS // NCHUNK, NCHUNK, CHUNK))
  g, selfb = _tc_mid(x, W, b.reshape(1, D), degp)
  parts = _sc_scatter(g, src2, dst2)
  return _tc_final(parts, degp, selfb)
